# Initial kernel scaffold; baseline (speedup 1.0000x reference)
import jax
import jax.numpy as jnp
from jax.experimental import pallas as pl


def kernel(x, embed_mu, embed_logvar):
    raise NotImplementedError("write your pallas kernel here")


# speedup vs baseline: 2.5983x; 2.5983x over previous
import functools

import jax
import jax.numpy as jnp
from jax import lax
from jax.experimental import pallas as pl
from jax.experimental.pallas import tpu as pltpu
from jax.experimental.pallas import tpu_sc as plsc

_BATCH = 16384
_HIST = 200
_D = 16
_STREAM = 128
_ROWS = (_BATCH * _HIST) // _STREAM
_NW = 32
_ROWS_PER_W = _ROWS // _NW
_NSTR = 8
_NITER = _ROWS_PER_W // _NSTR

_mesh = plsc.VectorSubcoreMesh(core_axis_name="c", subcore_axis_name="s")


@functools.partial(
    pl.kernel,
    mesh=_mesh,
    out_type=(
        jax.ShapeDtypeStruct((_ROWS, _STREAM, _D), jnp.float32),
        jax.ShapeDtypeStruct((_ROWS, _STREAM, _D), jnp.float32),
    ),
    scratch_types=[
        pltpu.VMEM((_NSTR, _STREAM), jnp.int32),
        pltpu.VMEM((_NSTR, _STREAM, _D), jnp.float32),
        pltpu.VMEM((_NSTR, _STREAM, _D), jnp.float32),
        pltpu.SemaphoreType.DMA,
        pltpu.SemaphoreType.DMA,
    ],
    compiler_params=pltpu.CompilerParams(use_tc_tiling_on_sc=False),
)
def _gather2(x_hbm, mu_hbm, lv_hbm, out_mu, out_lv,
             idx_v, mu_rows, lv_rows, sem_mu, sem_lv):
    cid = lax.axis_index("c")
    sid = lax.axis_index("s")
    wid = sid * 2 + cid
    row0 = wid * _ROWS_PER_W

    def body(j, carry):
        r = row0 + j * _NSTR
        pltpu.sync_copy(x_hbm.at[pl.ds(r, _NSTR)], idx_v)
        copies = []
        for t in range(_NSTR):
            copies.append(
                pltpu.async_copy(mu_hbm.at[idx_v.at[t]], mu_rows.at[t], sem_mu))
            copies.append(
                pltpu.async_copy(lv_hbm.at[idx_v.at[t]], lv_rows.at[t], sem_lv))
        for c in copies:
            c.wait()
        pltpu.sync_copy(mu_rows, out_mu.at[pl.ds(r, _NSTR)])
        pltpu.sync_copy(lv_rows, out_lv.at[pl.ds(r, _NSTR)])
        return carry

    lax.fori_loop(0, _NITER, body, 0)


def kernel(x, embed_mu, embed_logvar):
    x32 = x.astype(jnp.int32).reshape(_ROWS, _STREAM)
    out_mu, out_lv = _gather2(x32, embed_mu, embed_logvar)
    mu = out_mu.reshape(_BATCH, _HIST, _D)
    logvar = out_lv.reshape(_BATCH, _HIST, _D)
    return (mu, mu, logvar)


# speedup vs baseline: 2.6342x; 1.0138x over previous
import functools

import jax
import jax.numpy as jnp
from jax import lax
from jax.experimental import pallas as pl
from jax.experimental.pallas import tpu as pltpu
from jax.experimental.pallas import tpu_sc as plsc

_BATCH = 16384
_HIST = 200
_D = 16
_STREAM = 128
_ROWS = (_BATCH * _HIST) // _STREAM
_NW = 32
_ROWS_PER_W = _ROWS // _NW
_NSTR = 8
_NITER = _ROWS_PER_W // _NSTR

_mesh = plsc.VectorSubcoreMesh(core_axis_name="c", subcore_axis_name="s")


@functools.partial(
    pl.kernel,
    mesh=_mesh,
    out_type=(
        jax.ShapeDtypeStruct((_ROWS, _STREAM, _D), jnp.float32),
        jax.ShapeDtypeStruct((_ROWS, _STREAM, _D), jnp.float32),
    ),
    scratch_types=[
        pltpu.VMEM((2, _NSTR, _STREAM), jnp.int32),
        pltpu.VMEM((2, _NSTR, _STREAM, _D), jnp.float32),
        pltpu.VMEM((2, _NSTR, _STREAM, _D), jnp.float32),
        pltpu.SemaphoreType.DMA,
        pltpu.SemaphoreType.DMA,
        pltpu.SemaphoreType.DMA,
    ],
    compiler_params=pltpu.CompilerParams(use_tc_tiling_on_sc=False),
)
def _gather2(x_hbm, mu_hbm, lv_hbm, out_mu, out_lv,
             idx_v, mu_rows, lv_rows, sem_idx, sem_g, sem_w):
    cid = lax.axis_index("c")
    sid = lax.axis_index("s")
    wid = sid * 2 + cid
    row0 = wid * _ROWS_PER_W

    def fire_gathers(slot):
        for t in range(_NSTR):
            pltpu.async_copy(mu_hbm.at[idx_v.at[slot, t]],
                             mu_rows.at[slot, t], sem_g)
            pltpu.async_copy(lv_hbm.at[idx_v.at[slot, t]],
                             lv_rows.at[slot, t], sem_g)

    def drain_gathers(slot):
        for t in range(_NSTR):
            pltpu.make_async_copy(mu_hbm.at[idx_v.at[slot, t]],
                                  mu_rows.at[slot, t], sem_g).wait()
            pltpu.make_async_copy(lv_hbm.at[idx_v.at[slot, t]],
                                  lv_rows.at[slot, t], sem_g).wait()

    pltpu.sync_copy(x_hbm.at[pl.ds(row0, _NSTR)], idx_v.at[0])
    fire_gathers(0)

    def body(j, carry):
        s = j % 2
        ns = 1 - s
        r = row0 + j * _NSTR
        has_next = j + 1 < _NITER

        @pl.when(has_next)
        def _():
            pltpu.async_copy(x_hbm.at[pl.ds(r + _NSTR, _NSTR)],
                             idx_v.at[ns], sem_idx)

        drain_gathers(s)
        pltpu.async_copy(mu_rows.at[s], out_mu.at[pl.ds(r, _NSTR)], sem_w)
        pltpu.async_copy(lv_rows.at[s], out_lv.at[pl.ds(r, _NSTR)], sem_w)

        @pl.when(has_next & (j > 0))
        def _():
            pltpu.make_async_copy(mu_rows.at[ns],
                                  out_mu.at[pl.ds(r - _NSTR, _NSTR)],
                                  sem_w).wait()
            pltpu.make_async_copy(lv_rows.at[ns],
                                  out_lv.at[pl.ds(r - _NSTR, _NSTR)],
                                  sem_w).wait()

        @pl.when(has_next)
        def _():
            pltpu.make_async_copy(x_hbm.at[pl.ds(r + _NSTR, _NSTR)],
                                  idx_v.at[ns], sem_idx).wait()
            fire_gathers(ns)

        return carry

    lax.fori_loop(0, _NITER, body, 0)

    for jj in (_NITER - 2, _NITER - 1):
        s = jj % 2
        r = row0 + jj * _NSTR
        pltpu.make_async_copy(mu_rows.at[s],
                              out_mu.at[pl.ds(r, _NSTR)], sem_w).wait()
        pltpu.make_async_copy(lv_rows.at[s],
                              out_lv.at[pl.ds(r, _NSTR)], sem_w).wait()


def kernel(x, embed_mu, embed_logvar):
    x32 = x.astype(jnp.int32).reshape(_ROWS, _STREAM)
    out_mu, out_lv = _gather2(x32, embed_mu, embed_logvar)
    mu = out_mu.reshape(_BATCH, _HIST, _D)
    logvar = out_lv.reshape(_BATCH, _HIST, _D)
    return (mu, mu, logvar)
